# Initial kernel scaffold; baseline (speedup 1.0000x reference)
import jax
import jax.numpy as jnp
from jax.experimental import pallas as pl


def kernel(feat_data, adjs, fW_W, fW_b, a_src, a_dest, W0, b0, W1, b1, Wl, bl):
    raise NotImplementedError("write your pallas kernel here")


# speedup vs baseline: 2.2253x; 2.2253x over previous
import functools

import jax
import jax.numpy as jnp
from jax.experimental import pallas as pl
from jax.experimental.pallas import tpu as pltpu

N = 4096
NFEAT = 512
NHID = 128
NHEADS = 4
NOUT = 128
ALPHA = 0.2

BR = 256
BC = 1024
BRP = 256


def _mm2_kernel(x_ref, wa_ref, ba_ref, wb_ref, bb_ref, oa_ref, ob_ref):
    x = x_ref[...]
    oa_ref[...] = (
        jnp.dot(x, wa_ref[...], preferred_element_type=jnp.float32) + ba_ref[...]
    )
    ob_ref[...] = (
        jnp.dot(x, wb_ref[...], preferred_element_type=jnp.float32) + bb_ref[...]
    )


def _mm2(x, wa, ba, wb, bb):
    n, k = x.shape
    ma = wa.shape[1]
    mb = wb.shape[1]
    grid = (n // BRP,)
    return pl.pallas_call(
        _mm2_kernel,
        grid=grid,
        in_specs=[
            pl.BlockSpec((BRP, k), lambda r: (r, 0)),
            pl.BlockSpec((k, ma), lambda r: (0, 0)),
            pl.BlockSpec((1, ma), lambda r: (0, 0)),
            pl.BlockSpec((k, mb), lambda r: (0, 0)),
            pl.BlockSpec((1, mb), lambda r: (0, 0)),
        ],
        out_specs=[
            pl.BlockSpec((BRP, ma), lambda r: (r, 0)),
            pl.BlockSpec((BRP, mb), lambda r: (r, 0)),
        ],
        out_shape=[
            jax.ShapeDtypeStruct((n, ma), jnp.float32),
            jax.ShapeDtypeStruct((n, mb), jnp.float32),
        ],
    )(x, wa, ba, wb, bb)


def _stats_kernel(f_ref, rowv_ref, bv_ref):
    f = f_ref[...]
    f1 = f[:, 0:NHEADS]
    f2 = f[:, NHEADS : 2 * NHEADS]
    m2 = jnp.max(f2, axis=0, keepdims=True)
    t = f1 + m2
    m = jnp.maximum(t, ALPHA * t)
    rowv_ref[:, 0:NHEADS] = jnp.exp(t - m)
    rowv_ref[:, NHEADS : 2 * NHEADS] = jnp.exp(ALPHA * t - m)
    u = f2 - m2
    bv_ref[:, 0:NHEADS] = jnp.exp(u)
    bv_ref[:, NHEADS : 2 * NHEADS] = jnp.exp(ALPHA * u)


def _stats(f):
    return pl.pallas_call(
        _stats_kernel,
        out_shape=[
            jax.ShapeDtypeStruct((N, 2 * NHEADS), jnp.float32),
            jax.ShapeDtypeStruct((N, 2 * NHEADS), jnp.float32),
        ],
    )(f)


def _att_kernel(adj_ref, rowv_ref, colv_ref, v_ref, wl_ref, bl_ref,
                out_ref, u_scr, den_scr, *, final):
    c = pl.program_id(1)
    nc = pl.num_programs(1)

    @pl.when(c == 0)
    def _init():
        u_scr[...] = jnp.zeros_like(u_scr)
        den_scr[...] = jnp.zeros_like(den_scr)

    adj = adj_ref[...]
    for i in range(NHEADS):
        ap = rowv_ref[:, i : i + 1]
        an = rowv_ref[:, NHEADS + i : NHEADS + i + 1]
        bp = colv_ref[i : i + 1, :]
        bn = colv_ref[NHEADS + i : NHEADS + i + 1, :]
        p = jnp.maximum(ap * bp, an * bn) * adj
        vblk = v_ref[pl.ds(c * BC, BC), i * NHID : (i + 1) * NHID]
        u_scr[:, i * NHID : (i + 1) * NHID] += jnp.dot(
            p, vblk, preferred_element_type=jnp.float32
        )
        den_scr[:, i : i + 1] += jnp.sum(p, axis=1, keepdims=True)

    @pl.when(c == nc - 1)
    def _fin():
        u = u_scr[...]
        den = den_scr[...]
        cols = []
        for i in range(NHEADS):
            d = den[:, i : i + 1]
            ok = d > 0.0
            x = u[:, i * NHID : (i + 1) * NHID] / jnp.where(ok, d, 1.0)
            x = jnp.where(ok, x, 0.0)
            cols.append(jnp.where(x > 0.0, x, jnp.exp(x) - 1.0))
        x = jnp.concatenate(cols, axis=1)
        if final:
            out_ref[...] = (
                jnp.dot(x, wl_ref[...], preferred_element_type=jnp.float32)
                + bl_ref[...]
            )
        else:
            out_ref[...] = x


def _att_pass(adjs, rowv, colv, v, wl, bl, final):
    nout = NOUT if final else NHEADS * NHID
    grid = (N // BR, N // BC)
    return pl.pallas_call(
        functools.partial(_att_kernel, final=final),
        grid=grid,
        in_specs=[
            pl.BlockSpec((BR, BC), lambda r, c: (r, c)),
            pl.BlockSpec((BR, 2 * NHEADS), lambda r, c: (r, 0)),
            pl.BlockSpec((2 * NHEADS, BC), lambda r, c: (0, c)),
            pl.BlockSpec((N, NHEADS * NHID), lambda r, c: (0, 0)),
            pl.BlockSpec((NHEADS * NHID, NOUT), lambda r, c: (0, 0)),
            pl.BlockSpec((1, NOUT), lambda r, c: (0, 0)),
        ],
        out_specs=pl.BlockSpec((BR, nout), lambda r, c: (r, 0)),
        out_shape=jax.ShapeDtypeStruct((N, nout), jnp.float32),
        scratch_shapes=[
            pltpu.VMEM((BR, NHEADS * NHID), jnp.float32),
            pltpu.VMEM((BR, 2 * NHEADS), jnp.float32),
        ],
        compiler_params=pltpu.CompilerParams(
            dimension_semantics=("arbitrary", "arbitrary"),
        ),
    )(adjs, rowv, colv, v, wl, bl)


@jax.jit
def kernel(feat_data, adjs, fW_W, fW_b, a_src, a_dest, W0, b0, W1, b1, Wl, bl):
    w_src = jnp.einsum("hfk,hk->fh", fW_W, a_src)
    w_dst = jnp.einsum("hfk,hk->fh", fW_W, a_dest)
    wf = jnp.concatenate([w_src, w_dst], axis=1)
    cf = jnp.concatenate(
        [jnp.sum(fW_b * a_src, axis=1), jnp.sum(fW_b * a_dest, axis=1)]
    )[None, :]
    w0cat = jnp.concatenate(list(W0), axis=1)
    b0cat = jnp.concatenate(list(b0))[None, :]
    w1cat = jnp.concatenate(list(W1), axis=1)
    b1cat = jnp.concatenate(list(b1))[None, :]

    v0, f = _mm2(feat_data, w0cat, b0cat, wf, cf)
    rowv, bv = _stats(f)
    colv = bv.T

    x1 = _att_pass(adjs, rowv, colv, v0, Wl, bl[None, :], final=False)
    v1, _ = _mm2(x1, w1cat, b1cat, wf, cf)
    out = _att_pass(adjs, rowv, colv, v1, Wl, bl[None, :], final=True)
    return out


# speedup vs baseline: 2.3846x; 1.0716x over previous
import functools

import jax
import jax.numpy as jnp
from jax.experimental import pallas as pl
from jax.experimental.pallas import tpu as pltpu

N = 4096
NFEAT = 512
NHID = 128
NHEADS = 4
NOUT = 128
ALPHA = 0.2

BR = 256
BC = 1024
BRP = 256


def _mm2_kernel(x_ref, wa_ref, ba_ref, wb_ref, bb_ref, oa_ref, ob_ref):
    x = x_ref[...]
    oa_ref[...] = (
        jnp.dot(x, wa_ref[...], preferred_element_type=jnp.float32) + ba_ref[...]
    ).astype(jnp.bfloat16)
    ob_ref[...] = (
        jnp.dot(x, wb_ref[...], preferred_element_type=jnp.float32) + bb_ref[...]
    )


def _mm2(x, wa, ba, wb, bb):
    n, k = x.shape
    ma = wa.shape[1]
    mb = wb.shape[1]
    grid = (n // BRP,)
    return pl.pallas_call(
        _mm2_kernel,
        grid=grid,
        in_specs=[
            pl.BlockSpec((BRP, k), lambda r: (r, 0)),
            pl.BlockSpec((k, ma), lambda r: (0, 0)),
            pl.BlockSpec((1, ma), lambda r: (0, 0)),
            pl.BlockSpec((k, mb), lambda r: (0, 0)),
            pl.BlockSpec((1, mb), lambda r: (0, 0)),
        ],
        out_specs=[
            pl.BlockSpec((BRP, ma), lambda r: (r, 0)),
            pl.BlockSpec((BRP, mb), lambda r: (r, 0)),
        ],
        out_shape=[
            jax.ShapeDtypeStruct((n, ma), jnp.bfloat16),
            jax.ShapeDtypeStruct((n, mb), jnp.float32),
        ],
    )(x, wa, ba, wb, bb)


def _stats_kernel(f_ref, rowv_ref, bv_ref):
    f = f_ref[...]
    f1 = f[:, 0:NHEADS]
    f2 = f[:, NHEADS : 2 * NHEADS]
    m2 = jnp.max(f2, axis=0, keepdims=True)
    t = f1 + m2
    m = jnp.maximum(t, ALPHA * t)
    rowv_ref[:, 0:NHEADS] = jnp.exp(t - m)
    rowv_ref[:, NHEADS : 2 * NHEADS] = jnp.exp(ALPHA * t - m)
    u = f2 - m2
    bv_ref[:, 0:NHEADS] = jnp.exp(u)
    bv_ref[:, NHEADS : 2 * NHEADS] = jnp.exp(ALPHA * u)


def _stats(f):
    return pl.pallas_call(
        _stats_kernel,
        out_shape=[
            jax.ShapeDtypeStruct((N, 2 * NHEADS), jnp.float32),
            jax.ShapeDtypeStruct((N, 2 * NHEADS), jnp.float32),
        ],
    )(f)


def _map_and_dot(adj, rowv_ref, colv_ref, v_ref, u_scr, c):
    sums = []
    for i in range(NHEADS):
        ap = rowv_ref[:, i : i + 1]
        an = rowv_ref[:, NHEADS + i : NHEADS + i + 1]
        bp = colv_ref[i : i + 1, :]
        bn = colv_ref[NHEADS + i : NHEADS + i + 1, :]
        p = jnp.maximum(ap * bp, an * bn) * adj
        vblk = v_ref[pl.ds(c * BC, BC), i * NHID : (i + 1) * NHID]
        u_scr[:, i * NHID : (i + 1) * NHID] += jnp.dot(
            p.astype(jnp.bfloat16), vblk, preferred_element_type=jnp.float32
        )
        sums.append(p)
    return sums


def _norm_elu(u, den):
    cols = []
    for i in range(NHEADS):
        d = den[:, i : i + 1]
        ok = d > 0.0
        x = u[:, i * NHID : (i + 1) * NHID] / jnp.where(ok, d, 1.0)
        x = jnp.where(ok, x, 0.0)
        cols.append(jnp.where(x > 0.0, x, jnp.exp(x) - 1.0))
    return jnp.concatenate(cols, axis=1)


def _att1_kernel(adj_ref, rowv_ref, colv_ref, v_ref, out_ref, den_ref,
                 u_scr, den_scr):
    c = pl.program_id(1)
    nc = pl.num_programs(1)

    @pl.when(c == 0)
    def _init():
        u_scr[...] = jnp.zeros_like(u_scr)
        den_scr[...] = jnp.zeros_like(den_scr)

    ps = _map_and_dot(adj_ref[...], rowv_ref, colv_ref, v_ref, u_scr, c)
    for i in range(NHEADS):
        den_scr[:, i : i + 1] += jnp.sum(ps[i], axis=1, keepdims=True)

    @pl.when(c == nc - 1)
    def _fin():
        den_ref[...] = den_scr[...]
        out_ref[...] = _norm_elu(u_scr[...], den_scr[...]).astype(jnp.bfloat16)


def _att1_pass(adjs, rowv, colv, v):
    grid = (N // BR, N // BC)
    return pl.pallas_call(
        _att1_kernel,
        grid=grid,
        in_specs=[
            pl.BlockSpec((BR, BC), lambda r, c: (r, c)),
            pl.BlockSpec((BR, 2 * NHEADS), lambda r, c: (r, 0)),
            pl.BlockSpec((2 * NHEADS, BC), lambda r, c: (0, c)),
            pl.BlockSpec((N, NHEADS * NHID), lambda r, c: (0, 0)),
        ],
        out_specs=[
            pl.BlockSpec((BR, NHEADS * NHID), lambda r, c: (r, 0)),
            pl.BlockSpec((BR, 2 * NHEADS), lambda r, c: (r, 0)),
        ],
        out_shape=[
            jax.ShapeDtypeStruct((N, NHEADS * NHID), jnp.bfloat16),
            jax.ShapeDtypeStruct((N, 2 * NHEADS), jnp.float32),
        ],
        scratch_shapes=[
            pltpu.VMEM((BR, NHEADS * NHID), jnp.float32),
            pltpu.VMEM((BR, 2 * NHEADS), jnp.float32),
        ],
        compiler_params=pltpu.CompilerParams(
            dimension_semantics=("arbitrary", "arbitrary"),
        ),
    )(adjs, rowv, colv, v)


def _att2_kernel(adj_ref, rowv_ref, colv_ref, v_ref, den_ref, wl_ref, bl_ref,
                 out_ref, u_scr):
    c = pl.program_id(1)
    nc = pl.num_programs(1)

    @pl.when(c == 0)
    def _init():
        u_scr[...] = jnp.zeros_like(u_scr)

    _map_and_dot(adj_ref[...], rowv_ref, colv_ref, v_ref, u_scr, c)

    @pl.when(c == nc - 1)
    def _fin():
        x = _norm_elu(u_scr[...], den_ref[...])
        out_ref[...] = (
            jnp.dot(
                x.astype(jnp.bfloat16),
                wl_ref[...],
                preferred_element_type=jnp.float32,
            )
            + bl_ref[...]
        )


def _att2_pass(adjs, rowv, colv, v, den, wl, bl):
    grid = (N // BR, N // BC)
    return pl.pallas_call(
        _att2_kernel,
        grid=grid,
        in_specs=[
            pl.BlockSpec((BR, BC), lambda r, c: (r, c)),
            pl.BlockSpec((BR, 2 * NHEADS), lambda r, c: (r, 0)),
            pl.BlockSpec((2 * NHEADS, BC), lambda r, c: (0, c)),
            pl.BlockSpec((N, NHEADS * NHID), lambda r, c: (0, 0)),
            pl.BlockSpec((BR, 2 * NHEADS), lambda r, c: (r, 0)),
            pl.BlockSpec((NHEADS * NHID, NOUT), lambda r, c: (0, 0)),
            pl.BlockSpec((1, NOUT), lambda r, c: (0, 0)),
        ],
        out_specs=pl.BlockSpec((BR, NOUT), lambda r, c: (r, 0)),
        out_shape=jax.ShapeDtypeStruct((N, NOUT), jnp.float32),
        scratch_shapes=[
            pltpu.VMEM((BR, NHEADS * NHID), jnp.float32),
        ],
        compiler_params=pltpu.CompilerParams(
            dimension_semantics=("arbitrary", "arbitrary"),
        ),
    )(adjs, rowv, colv, v, den, wl, bl)


@jax.jit
def kernel(feat_data, adjs, fW_W, fW_b, a_src, a_dest, W0, b0, W1, b1, Wl, bl):
    w_src = jnp.einsum("hfk,hk->fh", fW_W, a_src)
    w_dst = jnp.einsum("hfk,hk->fh", fW_W, a_dest)
    wf = jnp.concatenate([w_src, w_dst], axis=1)
    cf = jnp.concatenate(
        [jnp.sum(fW_b * a_src, axis=1), jnp.sum(fW_b * a_dest, axis=1)]
    )[None, :]
    w0cat = jnp.concatenate(list(W0), axis=1)
    b0cat = jnp.concatenate(list(b0))[None, :]
    w1cat = jnp.concatenate(list(W1), axis=1)
    b1cat = jnp.concatenate(list(b1))[None, :]

    v0, f = _mm2(feat_data, w0cat, b0cat, wf, cf)
    rowv, bv = _stats(f)
    colv = bv.T

    x1, den = _att1_pass(adjs, rowv, colv, v0)
    v1, _ = _mm2(x1, w1cat, b1cat, wf, cf)
    out = _att2_pass(adjs, rowv, colv, v1, den, Wl, bl[None, :])
    return out
